# baseline (speedup 1.0000x reference)
import functools

import jax
import jax.numpy as jnp
from jax import lax
from jax.experimental import pallas as pl
from jax.experimental.pallas import tpu as pltpu
from jax.experimental.pallas import tpu_sc as plsc

EPS = 1e-5
K = 8
NC = 2
NS = 16
LANES = 16


def _round_bf16_bits(v):
    bits = lax.bitcast_convert_type(v, jnp.int32)
    rnd = bits + jnp.int32(0x7FFF) + (lax.shift_right_logical(bits, 16) & 1)
    return lax.shift_right_logical(rnd, 16)


def _router_body(HD, T, E, CAP, NB,
                 x_ref, rw_ref, w_ref, b_ref,
                 xn_ref, dest_ref, wv_ref, idx3_ref, lg_ref):
    i = pl.program_id(0)
    x = x_ref[...]
    var = jnp.mean(x * x, axis=1, keepdims=True)
    xn = x * lax.rsqrt(var + EPS) * rw_ref[...]
    lo = _round_bf16_bits(xn[:, :HD])
    hi = _round_bf16_bits(xn[:, HD:])
    xn_ref[...] = lo | lax.shift_left(hi, 16)
    rows = T // NB
    lg_ref[pl.ds(i * rows, rows), :] = (
        jnp.dot(xn, w_ref[...], preferred_element_type=jnp.float32) + b_ref[...]
    )

    @pl.when(i == NB - 1)
    def _route():
        _route_body(T, E, CAP, lg_ref, dest_ref, wv_ref, idx3_ref)


def _route_body(T, E, CAP, lg_ref, dest_ref, wv_ref, idx3_ref):
    scores = jax.nn.sigmoid(lg_ref[...])
    iota_e = lax.broadcasted_iota(jnp.int32, (T, E), 1)
    cur = scores
    topv, topi = [], []
    for _ in range(K):
        m = jnp.max(cur, axis=1, keepdims=True)
        ism = cur == m
        idx = jnp.min(jnp.where(ism, iota_e, E), axis=1, keepdims=True)
        topv.append(m)
        topi.append(idx)
        cur = jnp.where(iota_e == idx, -1.0, cur)
    tv = jnp.concatenate(topv, axis=1)
    wsum = jnp.sum(tv, axis=1, keepdims=True)
    wk = tv / wsum
    cnt = jnp.zeros((T, E), jnp.float32)
    for k in range(K):
        cnt = cnt + (iota_e == topi[k]).astype(jnp.float32)
    CH = 256
    r_i = lax.broadcasted_iota(jnp.int32, (CH, CH), 0)
    c_i = lax.broadcasted_iota(jnp.int32, (CH, CH), 1)
    tril = (c_i < r_i).astype(jnp.float32)
    parts = []
    carry = jnp.zeros((1, E), jnp.float32)
    for c in range(T // CH):
        blk = cnt[c * CH:(c + 1) * CH, :]
        parts.append(carry + jnp.dot(tril, blk, preferred_element_type=jnp.float32))
        carry = carry + jnp.sum(blk, axis=0, keepdims=True)
    prefix = jnp.concatenate(parts, axis=0)
    dests, wvs = [], []
    for k in range(K):
        sel = (iota_e == topi[k]).astype(jnp.float32)
        pos = jnp.sum(prefix * sel, axis=1, keepdims=True).astype(jnp.int32)
        keep = pos < CAP
        dests.append(jnp.where(keep, topi[k] * CAP + pos, E * CAP))
        wvs.append(jnp.where(keep, wk[:, k:k + 1], 0.0))
    dest_ref[...] = jnp.concatenate(dests, axis=1)
    wv_ref[...] = jnp.concatenate(wvs, axis=1)
    for k in range(K):
        idx3_ref[k] = dests[k].reshape(T // 16, 16)


def _dispatch_body(T, D, NSLOT, TK,
                   xn_hbm, dest_hbm, wv_hbm, idx3_hbm, buf_hbm, scale_hbm,
                   dest_ch, wv_ch, scale_v, idx_t, rows_v,
                   sem0, sem1, wsem):
    c = lax.axis_index("c")
    s_id = lax.axis_index("s")
    wid = c * NS + s_id
    n_init = NSLOT // LANES
    n_ch = TK // 2048

    CT = 16
    nck = (T // (NC * NS)) // CT
    tbase = wid * (T // (NC * NS))
    sems = (sem0, sem1)

    def _lstart(j, b):
        pltpu.async_copy(
            xn_hbm.at[pl.ds(tbase + j * CT, CT)], rows_v.at[b], sems[b]
        )

    def _lwait(j, b):
        pltpu.make_async_copy(
            xn_hbm.at[pl.ds(tbase + j * CT, CT)], rows_v.at[b], sems[b]
        ).wait()

    _lstart(0, 0)

    @pl.when((s_id == 0) & (c == 0))
    def _build():
        z_f = jnp.zeros((LANES,), jnp.float32)

        @pl.loop(0, n_init)
        def _init(i):
            scale_v[pl.ds(i * LANES, LANES)] = z_f

        @pl.loop(0, n_ch)
        def _chunk(cb):
            pltpu.sync_copy(dest_hbm.at[pl.ds(cb * 2048, 2048)], dest_ch)
            pltpu.sync_copy(wv_hbm.at[pl.ds(cb * 2048, 2048)], wv_ch)

            @pl.loop(0, 2048 // LANES)
            def _scat(i):
                d16 = dest_ch[pl.ds(i * LANES, LANES)]
                w16 = wv_ch[pl.ds(i * LANES, LANES)]
                plsc.store_scatter(scale_v, [d16], w16)

        pltpu.sync_copy(scale_v, scale_hbm)

    pltpu.sync_copy(idx3_hbm.at[:, pl.ds(wid * nck, nck), :], idx_t)

    def _scatter(j, b):
        _lwait(j, b)
        for k in range(K):
            pltpu.async_copy(rows_v.at[b], buf_hbm.at[idx_t.at[k, j]], wsem)
        for k in range(K):
            pltpu.make_async_copy(rows_v.at[b], buf_hbm.at[idx_t.at[k, j]], wsem).wait()

    @pl.loop(0, nck - 2, step=2)
    def _disp(j):
        for b in range(2):
            jj = j + b
            _lstart(jj + 1, 1 - b)
            _scatter(jj, b)

    _lstart(nck - 1, 1)
    _scatter(nck - 2, 0)
    _scatter(nck - 1, 1)


def _mlp_body(CAP, D, buf_ref, wg_ref, wu_ref, wd_ref, sc_ref, y_ref):
    w = buf_ref[...].reshape(CAP, D // 2)
    lo = lax.bitcast_convert_type(lax.shift_left(w, 16), jnp.float32)
    hi = lax.bitcast_convert_type(w & jnp.int32(-65536), jnp.float32)
    xg = jnp.concatenate([lo, hi], axis=1)
    g = jnp.dot(xg, wg_ref[0], preferred_element_type=jnp.float32)
    u = jnp.dot(xg, wu_ref[0], preferred_element_type=jnp.float32)
    a = g * jax.nn.sigmoid(g) * u
    y = jnp.dot(a, wd_ref[0], preferred_element_type=jnp.float32)
    scl = sc_ref[...]
    y = jnp.where(scl > 0.0, y * scl, 0.0)
    lo = _round_bf16_bits(y[:, :D // 2])
    hi = _round_bf16_bits(y[:, D // 2:])
    y_ref[...] = lo | lax.shift_left(hi, 16)


def _combine_body(T, D, y_hbm, dest_hbm, res_hbm, out_hbm,
                  idx_l, rows_v, acc_v, sem0, sem1):
    c = lax.axis_index("c")
    s_id = lax.axis_index("s")
    wid = c * NS + s_id
    tpt = T // (NC * NS)
    tbase = wid * tpt
    pltpu.sync_copy(dest_hbm.at[pl.ds(tbase * K, tpt * K)], idx_l)
    CT = 4
    nch = tpt // CT
    sems = (sem0, sem1)

    def _gstart(j, b):
        pltpu.async_copy(
            y_hbm.at[idx_l.at[pl.ds(j * CT * K, CT * K)]], rows_v.at[b], sems[b]
        )

    def _gwait(j, b):
        pltpu.make_async_copy(
            y_hbm.at[idx_l.at[pl.ds(j * CT * K, CT * K)]], rows_v.at[b], sems[b]
        ).wait()

    def _accum(j, b):
        cb = tbase + j * CT
        pltpu.sync_copy(res_hbm.at[pl.ds(cb, CT)], acc_v)
        _gwait(j, b)
        HD = D // 2

        @pl.loop(0, HD // LANES)
        def _vec(v):
            sl = pl.ds(v * LANES, LANES)
            sh = pl.ds(HD + v * LANES, LANES)
            for t in range(CT):
                a_lo = acc_v[t, sl]
                a_hi = acc_v[t, sh]
                for r in range(K):
                    w = rows_v[b, t * K + r, sl]
                    a_lo = a_lo + plsc.bitcast(lax.shift_left(w, 16), jnp.float32)
                    a_hi = a_hi + plsc.bitcast(w & jnp.int32(-65536), jnp.float32)
                acc_v[t, sl] = a_lo
                acc_v[t, sh] = a_hi

        pltpu.sync_copy(acc_v, out_hbm.at[pl.ds(cb, CT)])

    _gstart(0, 0)

    @pl.loop(0, nch - 2, step=2)
    def _chunk(j):
        for b in range(2):
            jj = j + b
            _gstart(jj + 1, 1 - b)
            _accum(jj, b)

    _gstart(nch - 1, 1)
    _accum(nch - 2, 0)
    _accum(nch - 1, 1)


def kernel(hidden_states, rms_weight, router_w, router_b, w_gate, w_up, w_down):
    b, s, d = hidden_states.shape
    T, D = b * s, d
    E = router_w.shape[1]
    F = w_gate.shape[2]
    CAP = T * K // E
    NSLOT = (E + 1) * CAP
    TK = T * K

    x = hidden_states.reshape(T, D)

    NB = 8
    xn, dest, wv, idx3, _lg = pl.pallas_call(
        functools.partial(_router_body, D // 2, T, E, CAP, NB),
        grid=(NB,),
        in_specs=[
            pl.BlockSpec((T // NB, D), lambda i: (i, 0)),
            pl.BlockSpec((1, D), lambda i: (0, 0)),
            pl.BlockSpec((D, E), lambda i: (0, 0)),
            pl.BlockSpec((1, E), lambda i: (0, 0)),
        ],
        out_specs=[
            pl.BlockSpec((T // NB, D // 2), lambda i: (i, 0)),
            pl.BlockSpec((T, K), lambda i: (0, 0)),
            pl.BlockSpec((T, K), lambda i: (0, 0)),
            pl.BlockSpec((K, T // 16, 16), lambda i: (0, 0, 0)),
            pl.BlockSpec((T, E), lambda i: (0, 0)),
        ],
        out_shape=[
            jax.ShapeDtypeStruct((T, D // 2), jnp.int32),
            jax.ShapeDtypeStruct((T, K), jnp.int32),
            jax.ShapeDtypeStruct((T, K), jnp.float32),
            jax.ShapeDtypeStruct((K, T // 16, 16), jnp.int32),
            jax.ShapeDtypeStruct((T, E), jnp.float32),
        ],
    )(x, rms_weight.reshape(1, D), router_w, router_b.reshape(1, E))
    xn3 = xn.reshape(T, 8, 128)

    dest_flat = dest.reshape(TK)
    wv_flat = wv.reshape(TK)

    mesh = plsc.VectorSubcoreMesh(core_axis_name="c", subcore_axis_name="s")
    buf, scale = pl.kernel(
        functools.partial(_dispatch_body, T, D, NSLOT, TK),
        out_type=[
            jax.ShapeDtypeStruct((NSLOT, 8, 128), jnp.int32),
            jax.ShapeDtypeStruct((NSLOT,), jnp.float32),
        ],
        mesh=mesh,
        scratch_types=[
            pltpu.VMEM((2048,), jnp.int32),
            pltpu.VMEM((2048,), jnp.float32),
            pltpu.VMEM((NSLOT,), jnp.float32),
            pltpu.VMEM((K, 4, 16), jnp.int32),
            pltpu.VMEM((2, 16, 8, 128), jnp.int32),
            pltpu.SemaphoreType.DMA,
            pltpu.SemaphoreType.DMA,
            pltpu.SemaphoreType.DMA,
        ],
        compiler_params=pltpu.CompilerParams(needs_layout_passes=False),
    )(xn3, dest_flat, wv_flat, idx3)

    nblk = E + 1
    y = pl.pallas_call(
        functools.partial(_mlp_body, CAP, D),
        grid=(nblk,),
        in_specs=[
            pl.BlockSpec((CAP, 8, 128), lambda e: (e, 0, 0)),
            pl.BlockSpec((1, D, F), lambda e: (jnp.minimum(e, E - 1), 0, 0)),
            pl.BlockSpec((1, D, F), lambda e: (jnp.minimum(e, E - 1), 0, 0)),
            pl.BlockSpec((1, F, D), lambda e: (jnp.minimum(e, E - 1), 0, 0)),
            pl.BlockSpec((CAP, 1), lambda e: (e, 0)),
        ],
        out_specs=pl.BlockSpec((CAP, D // 2), lambda e: (e, 0)),
        out_shape=jax.ShapeDtypeStruct((NSLOT, D // 2), jnp.int32),
    )(buf, w_gate, w_up, w_down, scale.reshape(NSLOT, 1))

    out = pl.kernel(
        functools.partial(_combine_body, T, D),
        out_type=jax.ShapeDtypeStruct((T, D), jnp.float32),
        mesh=mesh,
        scratch_types=[
            pltpu.VMEM((T // (NC * NS) * K,), jnp.int32),
            pltpu.VMEM((2, 4 * K, D // 2), jnp.int32),
            pltpu.VMEM((4, D), jnp.float32),
            pltpu.SemaphoreType.DMA,
            pltpu.SemaphoreType.DMA,
        ],
        compiler_params=pltpu.CompilerParams(needs_layout_passes=False),
    )(y, dest_flat, x)

    return out.reshape(b, s, d)


# speedup vs baseline: 1.0120x; 1.0120x over previous
import functools

import jax
import jax.numpy as jnp
from jax import lax
from jax.experimental import pallas as pl
from jax.experimental.pallas import tpu as pltpu
from jax.experimental.pallas import tpu_sc as plsc

EPS = 1e-5
K = 8
NC = 2
NS = 16
LANES = 16


def _round_bf16_bits(v):
    bits = lax.bitcast_convert_type(v, jnp.int32)
    rnd = bits + jnp.int32(0x7FFF) + (lax.shift_right_logical(bits, 16) & 1)
    return lax.shift_right_logical(rnd, 16)


def _router_body(HD, T, E, CAP, NB,
                 x_ref, rw_ref, w_ref, b_ref,
                 xn_ref, dest_ref, wv_ref, lg_ref):
    i = pl.program_id(0)
    x = x_ref[...]
    var = jnp.mean(x * x, axis=1, keepdims=True)
    xn = x * lax.rsqrt(var + EPS) * rw_ref[...]
    lo = _round_bf16_bits(xn[:, :HD])
    hi = _round_bf16_bits(xn[:, HD:])
    xn_ref[...] = lo | lax.shift_left(hi, 16)
    rows = T // NB
    lg_ref[pl.ds(i * rows, rows), :] = (
        jnp.dot(xn, w_ref[...], preferred_element_type=jnp.float32) + b_ref[...]
    )

    @pl.when(i == NB - 1)
    def _route():
        _route_body(T, E, CAP, lg_ref, dest_ref, wv_ref)


def _route_body(T, E, CAP, lg_ref, dest_ref, wv_ref):
    scores = jax.nn.sigmoid(lg_ref[...])
    iota_e = lax.broadcasted_iota(jnp.int32, (T, E), 1)
    cur = scores
    topv, topi = [], []
    for _ in range(K):
        m = jnp.max(cur, axis=1, keepdims=True)
        ism = cur == m
        idx = jnp.min(jnp.where(ism, iota_e, E), axis=1, keepdims=True)
        topv.append(m)
        topi.append(idx)
        cur = jnp.where(iota_e == idx, -1.0, cur)
    tv = jnp.concatenate(topv, axis=1)
    wsum = jnp.sum(tv, axis=1, keepdims=True)
    wk = tv / wsum
    cnt = jnp.zeros((T, E), jnp.float32)
    for k in range(K):
        cnt = cnt + (iota_e == topi[k]).astype(jnp.float32)
    CH = 256
    r_i = lax.broadcasted_iota(jnp.int32, (CH, CH), 0)
    c_i = lax.broadcasted_iota(jnp.int32, (CH, CH), 1)
    tril = (c_i < r_i).astype(jnp.float32)
    parts = []
    carry = jnp.zeros((1, E), jnp.float32)
    for c in range(T // CH):
        blk = cnt[c * CH:(c + 1) * CH, :]
        parts.append(carry + jnp.dot(tril, blk, preferred_element_type=jnp.float32))
        carry = carry + jnp.sum(blk, axis=0, keepdims=True)
    prefix = jnp.concatenate(parts, axis=0)
    dests, wvs = [], []
    for k in range(K):
        sel = (iota_e == topi[k]).astype(jnp.float32)
        pos = jnp.sum(prefix * sel, axis=1, keepdims=True).astype(jnp.int32)
        keep = pos < CAP
        dests.append(jnp.where(keep, topi[k] * CAP + pos, E * CAP))
        wvs.append(jnp.where(keep, wk[:, k:k + 1], 0.0))
    dest_ref[...] = jnp.concatenate(dests, axis=1)
    wv_ref[...] = jnp.concatenate(wvs, axis=1)


def _dispatch_body(T, D, NSLOT, TK,
                   xn_hbm, dest_hbm, wv_hbm, idx3_hbm, buf_hbm, scale_hbm,
                   dest_ch, wv_ch, scale_v, idx_t, rows_v,
                   sem0, sem1, wsem):
    c = lax.axis_index("c")
    s_id = lax.axis_index("s")
    wid = c * NS + s_id
    n_init = NSLOT // LANES
    n_ch = TK // 2048

    CT = 16
    nck = (T // (NC * NS)) // CT
    tbase = wid * (T // (NC * NS))
    sems = (sem0, sem1)

    def _lstart(j, b):
        pltpu.async_copy(
            xn_hbm.at[pl.ds(tbase + j * CT, CT)], rows_v.at[b], sems[b]
        )

    def _lwait(j, b):
        pltpu.make_async_copy(
            xn_hbm.at[pl.ds(tbase + j * CT, CT)], rows_v.at[b], sems[b]
        ).wait()

    _lstart(0, 0)

    @pl.when((s_id == 0) & (c == 0))
    def _build():
        z_f = jnp.zeros((LANES,), jnp.float32)

        @pl.loop(0, n_init)
        def _init(i):
            scale_v[pl.ds(i * LANES, LANES)] = z_f

        @pl.loop(0, n_ch)
        def _chunk(cb):
            pltpu.sync_copy(dest_hbm.at[pl.ds(cb * 2048, 2048)], dest_ch)
            pltpu.sync_copy(wv_hbm.at[pl.ds(cb * 2048, 2048)], wv_ch)

            @pl.loop(0, 2048 // LANES)
            def _scat(i):
                d16 = dest_ch[pl.ds(i * LANES, LANES)]
                w16 = wv_ch[pl.ds(i * LANES, LANES)]
                plsc.store_scatter(scale_v, [d16], w16)

        pltpu.sync_copy(scale_v, scale_hbm)

    pltpu.sync_copy(idx3_hbm.at[:, pl.ds(wid * nck, nck), :], idx_t)

    def _scatter(j, b):
        _lwait(j, b)
        for k in range(K):
            pltpu.async_copy(rows_v.at[b], buf_hbm.at[idx_t.at[k, j]], wsem)
        for k in range(K):
            pltpu.make_async_copy(rows_v.at[b], buf_hbm.at[idx_t.at[k, j]], wsem).wait()

    @pl.loop(0, nck - 2, step=2)
    def _disp(j):
        for b in range(2):
            jj = j + b
            _lstart(jj + 1, 1 - b)
            _scatter(jj, b)

    _lstart(nck - 1, 1)
    _scatter(nck - 2, 0)
    _scatter(nck - 1, 1)


def _mlp_body(CAP, D, buf_ref, wg_ref, wu_ref, wd_ref, sc_ref, y_ref):
    w = buf_ref[...].reshape(CAP, D // 2)
    lo = lax.bitcast_convert_type(lax.shift_left(w, 16), jnp.float32)
    hi = lax.bitcast_convert_type(w & jnp.int32(-65536), jnp.float32)
    xg = jnp.concatenate([lo, hi], axis=1)
    g = jnp.dot(xg, wg_ref[0], preferred_element_type=jnp.float32)
    u = jnp.dot(xg, wu_ref[0], preferred_element_type=jnp.float32)
    a = g * jax.nn.sigmoid(g) * u
    y = jnp.dot(a, wd_ref[0], preferred_element_type=jnp.float32)
    scl = sc_ref[...]
    y = jnp.where(scl > 0.0, y * scl, 0.0)
    lo = _round_bf16_bits(y[:, :D // 2])
    hi = _round_bf16_bits(y[:, D // 2:])
    y_ref[...] = lo | lax.shift_left(hi, 16)


def _combine_body(T, D, y_hbm, dest_hbm, res_hbm, out_hbm,
                  idx_l, rows_v, acc_v, sem0, sem1):
    c = lax.axis_index("c")
    s_id = lax.axis_index("s")
    wid = c * NS + s_id
    tpt = T // (NC * NS)
    tbase = wid * tpt
    pltpu.sync_copy(dest_hbm.at[pl.ds(tbase * K, tpt * K)], idx_l)
    CT = 4
    nch = tpt // CT
    sems = (sem0, sem1)

    def _gstart(j, b):
        pltpu.async_copy(
            y_hbm.at[idx_l.at[pl.ds(j * CT * K, CT * K)]], rows_v.at[b], sems[b]
        )

    def _gwait(j, b):
        pltpu.make_async_copy(
            y_hbm.at[idx_l.at[pl.ds(j * CT * K, CT * K)]], rows_v.at[b], sems[b]
        ).wait()

    def _accum(j, b):
        cb = tbase + j * CT
        pltpu.sync_copy(res_hbm.at[pl.ds(cb, CT)], acc_v)
        _gwait(j, b)
        HD = D // 2

        @pl.loop(0, HD // LANES)
        def _vec(v):
            sl = pl.ds(v * LANES, LANES)
            sh = pl.ds(HD + v * LANES, LANES)
            for t in range(CT):
                a_lo = acc_v[t, sl]
                a_hi = acc_v[t, sh]
                for r in range(K):
                    w = rows_v[b, t * K + r, sl]
                    a_lo = a_lo + plsc.bitcast(lax.shift_left(w, 16), jnp.float32)
                    a_hi = a_hi + plsc.bitcast(w & jnp.int32(-65536), jnp.float32)
                acc_v[t, sl] = a_lo
                acc_v[t, sh] = a_hi

        pltpu.sync_copy(acc_v, out_hbm.at[pl.ds(cb, CT)])

    _gstart(0, 0)

    @pl.loop(0, nch - 2, step=2)
    def _chunk(j):
        for b in range(2):
            jj = j + b
            _gstart(jj + 1, 1 - b)
            _accum(jj, b)

    _gstart(nch - 1, 1)
    _accum(nch - 2, 0)
    _accum(nch - 1, 1)


def kernel(hidden_states, rms_weight, router_w, router_b, w_gate, w_up, w_down):
    b, s, d = hidden_states.shape
    T, D = b * s, d
    E = router_w.shape[1]
    F = w_gate.shape[2]
    CAP = T * K // E
    NSLOT = (E + 1) * CAP
    TK = T * K

    x = hidden_states.reshape(T, D)

    NB = 8
    xn, dest, wv, _lg = pl.pallas_call(
        functools.partial(_router_body, D // 2, T, E, CAP, NB),
        grid=(NB,),
        in_specs=[
            pl.BlockSpec((T // NB, D), lambda i: (i, 0)),
            pl.BlockSpec((1, D), lambda i: (0, 0)),
            pl.BlockSpec((D, E), lambda i: (0, 0)),
            pl.BlockSpec((1, E), lambda i: (0, 0)),
        ],
        out_specs=[
            pl.BlockSpec((T // NB, D // 2), lambda i: (i, 0)),
            pl.BlockSpec((T, K), lambda i: (0, 0)),
            pl.BlockSpec((T, K), lambda i: (0, 0)),
            pl.BlockSpec((T, E), lambda i: (0, 0)),
        ],
        out_shape=[
            jax.ShapeDtypeStruct((T, D // 2), jnp.int32),
            jax.ShapeDtypeStruct((T, K), jnp.int32),
            jax.ShapeDtypeStruct((T, K), jnp.float32),
            jax.ShapeDtypeStruct((T, E), jnp.float32),
        ],
    )(x, rms_weight.reshape(1, D), router_w, router_b.reshape(1, E))
    xn3 = xn.reshape(T, 8, 128)

    dest_flat = dest.reshape(TK)
    wv_flat = wv.reshape(TK)
    idx3 = dest.reshape(T // 16, 16, K).transpose(2, 0, 1)

    mesh = plsc.VectorSubcoreMesh(core_axis_name="c", subcore_axis_name="s")
    buf, scale = pl.kernel(
        functools.partial(_dispatch_body, T, D, NSLOT, TK),
        out_type=[
            jax.ShapeDtypeStruct((NSLOT, 8, 128), jnp.int32),
            jax.ShapeDtypeStruct((NSLOT,), jnp.float32),
        ],
        mesh=mesh,
        scratch_types=[
            pltpu.VMEM((2048,), jnp.int32),
            pltpu.VMEM((2048,), jnp.float32),
            pltpu.VMEM((NSLOT,), jnp.float32),
            pltpu.VMEM((K, 4, 16), jnp.int32),
            pltpu.VMEM((2, 16, 8, 128), jnp.int32),
            pltpu.SemaphoreType.DMA,
            pltpu.SemaphoreType.DMA,
            pltpu.SemaphoreType.DMA,
        ],
        compiler_params=pltpu.CompilerParams(needs_layout_passes=False),
    )(xn3, dest_flat, wv_flat, idx3)

    nblk = E + 1
    y = pl.pallas_call(
        functools.partial(_mlp_body, CAP, D),
        grid=(nblk,),
        in_specs=[
            pl.BlockSpec((CAP, 8, 128), lambda e: (e, 0, 0)),
            pl.BlockSpec((1, D, F), lambda e: (jnp.minimum(e, E - 1), 0, 0)),
            pl.BlockSpec((1, D, F), lambda e: (jnp.minimum(e, E - 1), 0, 0)),
            pl.BlockSpec((1, F, D), lambda e: (jnp.minimum(e, E - 1), 0, 0)),
            pl.BlockSpec((CAP, 1), lambda e: (e, 0)),
        ],
        out_specs=pl.BlockSpec((CAP, D // 2), lambda e: (e, 0)),
        out_shape=jax.ShapeDtypeStruct((NSLOT, D // 2), jnp.int32),
    )(buf, w_gate, w_up, w_down, scale.reshape(NSLOT, 1))

    out = pl.kernel(
        functools.partial(_combine_body, T, D),
        out_type=jax.ShapeDtypeStruct((T, D), jnp.float32),
        mesh=mesh,
        scratch_types=[
            pltpu.VMEM((T // (NC * NS) * K,), jnp.int32),
            pltpu.VMEM((2, 4 * K, D // 2), jnp.int32),
            pltpu.VMEM((4, D), jnp.float32),
            pltpu.SemaphoreType.DMA,
            pltpu.SemaphoreType.DMA,
        ],
        compiler_params=pltpu.CompilerParams(needs_layout_passes=False),
    )(y, dest_flat, x)

    return out.reshape(b, s, d)


# speedup vs baseline: 1.0177x; 1.0056x over previous
import functools

import jax
import jax.numpy as jnp
from jax import lax
from jax.experimental import pallas as pl
from jax.experimental.pallas import tpu as pltpu
from jax.experimental.pallas import tpu_sc as plsc

EPS = 1e-5
K = 8
NC = 2
NS = 16
LANES = 16


def _round_bf16_bits(v):
    bits = lax.bitcast_convert_type(v, jnp.int32)
    rnd = bits + jnp.int32(0x7FFF) + (lax.shift_right_logical(bits, 16) & 1)
    return lax.shift_right_logical(rnd, 16)


def _norm_router_body(HD, x_ref, rw_ref, w_ref, b_ref, xn_ref, lg_ref):
    x = x_ref[...]
    var = jnp.mean(x * x, axis=1, keepdims=True)
    xn = x * lax.rsqrt(var + EPS) * rw_ref[...]
    lo = _round_bf16_bits(xn[:, :HD])
    hi = _round_bf16_bits(xn[:, HD:])
    xn_ref[...] = lo | lax.shift_left(hi, 16)
    lg_ref[...] = (
        jnp.dot(xn, w_ref[...], preferred_element_type=jnp.float32) + b_ref[...]
    )


def _route_body(T, E, CAP, lg_ref, dest_ref, wv_ref):
    scores = jax.nn.sigmoid(lg_ref[...])
    iota_e = lax.broadcasted_iota(jnp.int32, (T, E), 1)
    cur = scores
    topv, topi = [], []
    for _ in range(K):
        m = jnp.max(cur, axis=1, keepdims=True)
        ism = cur == m
        idx = jnp.min(jnp.where(ism, iota_e, E), axis=1, keepdims=True)
        topv.append(m)
        topi.append(idx)
        cur = jnp.where(iota_e == idx, -1.0, cur)
    tv = jnp.concatenate(topv, axis=1)
    wsum = jnp.sum(tv, axis=1, keepdims=True)
    wk = tv / wsum
    cnt = jnp.zeros((T, E), jnp.float32)
    for k in range(K):
        cnt = cnt + (iota_e == topi[k]).astype(jnp.float32)
    CH = 256
    r_i = lax.broadcasted_iota(jnp.int32, (CH, CH), 0)
    c_i = lax.broadcasted_iota(jnp.int32, (CH, CH), 1)
    tril = (c_i < r_i).astype(jnp.float32)
    parts = []
    carry = jnp.zeros((1, E), jnp.float32)
    for c in range(T // CH):
        blk = cnt[c * CH:(c + 1) * CH, :]
        parts.append(carry + jnp.dot(tril, blk, preferred_element_type=jnp.float32))
        carry = carry + jnp.sum(blk, axis=0, keepdims=True)
    prefix = jnp.concatenate(parts, axis=0)
    dests, wvs = [], []
    for k in range(K):
        sel = (iota_e == topi[k]).astype(jnp.float32)
        pos = jnp.sum(prefix * sel, axis=1, keepdims=True).astype(jnp.int32)
        keep = pos < CAP
        dests.append(jnp.where(keep, topi[k] * CAP + pos, E * CAP))
        wvs.append(jnp.where(keep, wk[:, k:k + 1], 0.0))
    dest_ref[...] = jnp.concatenate(dests, axis=1)
    wv_ref[...] = jnp.concatenate(wvs, axis=1)


def _dispatch_body(T, D, NSLOT, TK,
                   xn_hbm, dest_hbm, wv_hbm, idx3_hbm, buf_hbm, scale_hbm,
                   dest_ch, wv_ch, scale_v, idx_t, rows_v,
                   sem0, sem1, wsem):
    c = lax.axis_index("c")
    s_id = lax.axis_index("s")
    wid = c * NS + s_id
    n_init = NSLOT // LANES
    n_ch = TK // 2048

    CT = 16
    nck = (T // (NC * NS)) // CT
    tbase = wid * (T // (NC * NS))
    sems = (sem0, sem1)

    def _lstart(j, b):
        pltpu.async_copy(
            xn_hbm.at[pl.ds(tbase + j * CT, CT)], rows_v.at[b], sems[b]
        )

    def _lwait(j, b):
        pltpu.make_async_copy(
            xn_hbm.at[pl.ds(tbase + j * CT, CT)], rows_v.at[b], sems[b]
        ).wait()

    _lstart(0, 0)

    @pl.when((s_id == 0) & (c == 0))
    def _build():
        z_f = jnp.zeros((LANES,), jnp.float32)

        @pl.loop(0, n_init)
        def _init(i):
            scale_v[pl.ds(i * LANES, LANES)] = z_f

        @pl.loop(0, n_ch)
        def _chunk(cb):
            pltpu.sync_copy(dest_hbm.at[pl.ds(cb * 2048, 2048)], dest_ch)
            pltpu.sync_copy(wv_hbm.at[pl.ds(cb * 2048, 2048)], wv_ch)

            @pl.loop(0, 2048 // LANES)
            def _scat(i):
                d16 = dest_ch[pl.ds(i * LANES, LANES)]
                w16 = wv_ch[pl.ds(i * LANES, LANES)]
                plsc.store_scatter(scale_v, [d16], w16)

        pltpu.sync_copy(scale_v, scale_hbm)

    pltpu.sync_copy(idx3_hbm.at[:, pl.ds(wid * nck, nck), :], idx_t)

    def _scatter(j, b):
        _lwait(j, b)
        for k in range(K):
            pltpu.async_copy(rows_v.at[b], buf_hbm.at[idx_t.at[k, j]], wsem)
        for k in range(K):
            pltpu.make_async_copy(rows_v.at[b], buf_hbm.at[idx_t.at[k, j]], wsem).wait()

    @pl.loop(0, nck - 2, step=2)
    def _disp(j):
        for b in range(2):
            jj = j + b
            _lstart(jj + 1, 1 - b)
            _scatter(jj, b)

    _lstart(nck - 1, 1)
    _scatter(nck - 2, 0)
    _scatter(nck - 1, 1)


def _mlp_body(CAP, D, buf_ref, wg_ref, wu_ref, wd_ref, sc_ref, y_ref):
    w = buf_ref[...].reshape(CAP, D // 2)
    lo = lax.bitcast_convert_type(lax.shift_left(w, 16), jnp.float32)
    hi = lax.bitcast_convert_type(w & jnp.int32(-65536), jnp.float32)
    xg = jnp.concatenate([lo, hi], axis=1)
    g = jnp.dot(xg, wg_ref[0], preferred_element_type=jnp.float32)
    u = jnp.dot(xg, wu_ref[0], preferred_element_type=jnp.float32)
    a = g * jax.nn.sigmoid(g) * u
    y = jnp.dot(a, wd_ref[0], preferred_element_type=jnp.float32)
    scl = sc_ref[...]
    y = jnp.where(scl > 0.0, y * scl, 0.0)
    lo = _round_bf16_bits(y[:, :D // 2])
    hi = _round_bf16_bits(y[:, D // 2:])
    y_ref[...] = lo | lax.shift_left(hi, 16)


def _combine_body(T, D, y_hbm, dest_hbm, res_hbm, out_hbm,
                  idx_l, rows_v, acc_v, sem0, sem1):
    c = lax.axis_index("c")
    s_id = lax.axis_index("s")
    wid = c * NS + s_id
    tpt = T // (NC * NS)
    tbase = wid * tpt
    pltpu.sync_copy(dest_hbm.at[pl.ds(tbase * K, tpt * K)], idx_l)
    CT = 4
    nch = tpt // CT
    sems = (sem0, sem1)

    def _gstart(j, b):
        pltpu.async_copy(
            y_hbm.at[idx_l.at[pl.ds(j * CT * K, CT * K)]], rows_v.at[b], sems[b]
        )

    def _gwait(j, b):
        pltpu.make_async_copy(
            y_hbm.at[idx_l.at[pl.ds(j * CT * K, CT * K)]], rows_v.at[b], sems[b]
        ).wait()

    def _accum(j, b):
        cb = tbase + j * CT
        pltpu.sync_copy(res_hbm.at[pl.ds(cb, CT)], acc_v)
        _gwait(j, b)
        HD = D // 2

        @pl.loop(0, HD // LANES)
        def _vec(v):
            sl = pl.ds(v * LANES, LANES)
            sh = pl.ds(HD + v * LANES, LANES)
            for t in range(CT):
                a_lo = acc_v[t, sl]
                a_hi = acc_v[t, sh]
                for r in range(K):
                    w = rows_v[b, t * K + r, sl]
                    a_lo = a_lo + plsc.bitcast(lax.shift_left(w, 16), jnp.float32)
                    a_hi = a_hi + plsc.bitcast(w & jnp.int32(-65536), jnp.float32)
                acc_v[t, sl] = a_lo
                acc_v[t, sh] = a_hi

        pltpu.sync_copy(acc_v, out_hbm.at[pl.ds(cb, CT)])

    _gstart(0, 0)

    @pl.loop(0, nch - 2, step=2)
    def _chunk(j):
        for b in range(2):
            jj = j + b
            _gstart(jj + 1, 1 - b)
            _accum(jj, b)

    _gstart(nch - 1, 1)
    _accum(nch - 2, 0)
    _accum(nch - 1, 1)


def kernel(hidden_states, rms_weight, router_w, router_b, w_gate, w_up, w_down):
    b, s, d = hidden_states.shape
    T, D = b * s, d
    E = router_w.shape[1]
    F = w_gate.shape[2]
    CAP = T * K // E
    NSLOT = (E + 1) * CAP
    TK = T * K

    x = hidden_states.reshape(T, D)

    NB = 8
    xn, lg = pl.pallas_call(
        functools.partial(_norm_router_body, D // 2),
        grid=(NB,),
        in_specs=[
            pl.BlockSpec((T // NB, D), lambda i: (i, 0)),
            pl.BlockSpec((1, D), lambda i: (0, 0)),
            pl.BlockSpec((D, E), lambda i: (0, 0)),
            pl.BlockSpec((1, E), lambda i: (0, 0)),
        ],
        out_specs=[
            pl.BlockSpec((T // NB, D // 2), lambda i: (i, 0)),
            pl.BlockSpec((T // NB, E), lambda i: (i, 0)),
        ],
        out_shape=[
            jax.ShapeDtypeStruct((T, D // 2), jnp.int32),
            jax.ShapeDtypeStruct((T, E), jnp.float32),
        ],
    )(x, rms_weight.reshape(1, D), router_w, router_b.reshape(1, E))
    xn3 = xn.reshape(T, 8, 128)

    dest, wv = pl.pallas_call(
        functools.partial(_route_body, T, E, CAP),
        out_shape=[
            jax.ShapeDtypeStruct((T, K), jnp.int32),
            jax.ShapeDtypeStruct((T, K), jnp.float32),
        ],
    )(lg)

    dest_flat = dest.reshape(TK)
    wv_flat = wv.reshape(TK)
    idx3 = dest.reshape(T // 16, 16, K).transpose(2, 0, 1)

    mesh = plsc.VectorSubcoreMesh(core_axis_name="c", subcore_axis_name="s")
    buf, scale = pl.kernel(
        functools.partial(_dispatch_body, T, D, NSLOT, TK),
        out_type=[
            jax.ShapeDtypeStruct((NSLOT, 8, 128), jnp.int32),
            jax.ShapeDtypeStruct((NSLOT,), jnp.float32),
        ],
        mesh=mesh,
        scratch_types=[
            pltpu.VMEM((2048,), jnp.int32),
            pltpu.VMEM((2048,), jnp.float32),
            pltpu.VMEM((NSLOT,), jnp.float32),
            pltpu.VMEM((K, 4, 16), jnp.int32),
            pltpu.VMEM((2, 16, 8, 128), jnp.int32),
            pltpu.SemaphoreType.DMA,
            pltpu.SemaphoreType.DMA,
            pltpu.SemaphoreType.DMA,
        ],
        compiler_params=pltpu.CompilerParams(needs_layout_passes=False),
    )(xn3, dest_flat, wv_flat, idx3)

    nblk = E + 1
    y = pl.pallas_call(
        functools.partial(_mlp_body, CAP, D),
        grid=(nblk,),
        in_specs=[
            pl.BlockSpec((CAP, 8, 128), lambda e: (e, 0, 0)),
            pl.BlockSpec((1, D, F), lambda e: (jnp.minimum(e, E - 1), 0, 0)),
            pl.BlockSpec((1, D, F), lambda e: (jnp.minimum(e, E - 1), 0, 0)),
            pl.BlockSpec((1, F, D), lambda e: (jnp.minimum(e, E - 1), 0, 0)),
            pl.BlockSpec((CAP, 1), lambda e: (e, 0)),
        ],
        out_specs=pl.BlockSpec((CAP, D // 2), lambda e: (e, 0)),
        out_shape=jax.ShapeDtypeStruct((NSLOT, D // 2), jnp.int32),
    )(buf, w_gate, w_up, w_down, scale.reshape(NSLOT, 1))

    out = pl.kernel(
        functools.partial(_combine_body, T, D),
        out_type=jax.ShapeDtypeStruct((T, D), jnp.float32),
        mesh=mesh,
        scratch_types=[
            pltpu.VMEM((T // (NC * NS) * K,), jnp.int32),
            pltpu.VMEM((2, 4 * K, D // 2), jnp.int32),
            pltpu.VMEM((4, D), jnp.float32),
            pltpu.SemaphoreType.DMA,
            pltpu.SemaphoreType.DMA,
        ],
        compiler_params=pltpu.CompilerParams(needs_layout_passes=False),
    )(y, dest_flat, x)

    return out.reshape(b, s, d)


# speedup vs baseline: 1.0250x; 1.0072x over previous
import functools

import jax
import jax.numpy as jnp
from jax import lax
from jax.experimental import pallas as pl
from jax.experimental.pallas import tpu as pltpu
from jax.experimental.pallas import tpu_sc as plsc

EPS = 1e-5
K = 8
NC = 2
NS = 16
LANES = 16


def _round_bf16_bits(v):
    bits = lax.bitcast_convert_type(v, jnp.int32)
    rnd = bits + jnp.int32(0x7FFF) + (lax.shift_right_logical(bits, 16) & 1)
    return lax.shift_right_logical(rnd, 16)


def _norm_router_body(HD, x_ref, rw_ref, w_ref, b_ref, xn_ref, lg_ref):
    x = x_ref[...]
    var = jnp.mean(x * x, axis=1, keepdims=True)
    xn = x * lax.rsqrt(var + EPS) * rw_ref[...]
    lo = _round_bf16_bits(xn[:, :HD])
    hi = _round_bf16_bits(xn[:, HD:])
    xn_ref[...] = lo | lax.shift_left(hi, 16)
    lg_ref[...] = (
        jnp.dot(xn, w_ref[...], preferred_element_type=jnp.float32) + b_ref[...]
    )


def _route_body(T, E, CAP, lg_ref, dest_ref, wv_ref):
    scores = jax.nn.sigmoid(lg_ref[...])
    iota_e = lax.broadcasted_iota(jnp.int32, (T, E), 1)
    cur = scores
    topv, topi = [], []
    for _ in range(K):
        m = jnp.max(cur, axis=1, keepdims=True)
        ism = cur == m
        idx = jnp.min(jnp.where(ism, iota_e, E), axis=1, keepdims=True)
        topv.append(m)
        topi.append(idx)
        cur = jnp.where(iota_e == idx, -1.0, cur)
    tv = jnp.concatenate(topv, axis=1)
    wsum = jnp.sum(tv, axis=1, keepdims=True)
    wk = tv / wsum
    cnt = jnp.zeros((T, E), jnp.float32)
    for k in range(K):
        cnt = cnt + (iota_e == topi[k]).astype(jnp.float32)
    CH = 256
    r_i = lax.broadcasted_iota(jnp.int32, (CH, CH), 0)
    c_i = lax.broadcasted_iota(jnp.int32, (CH, CH), 1)
    tril = (c_i < r_i).astype(jnp.float32)
    parts = []
    carry = jnp.zeros((1, E), jnp.float32)
    for c in range(T // CH):
        blk = cnt[c * CH:(c + 1) * CH, :]
        parts.append(carry + jnp.dot(tril, blk, preferred_element_type=jnp.float32))
        carry = carry + jnp.sum(blk, axis=0, keepdims=True)
    prefix = jnp.concatenate(parts, axis=0)
    dests, wvs = [], []
    for k in range(K):
        sel = (iota_e == topi[k]).astype(jnp.float32)
        pos = jnp.sum(prefix * sel, axis=1, keepdims=True).astype(jnp.int32)
        keep = pos < CAP
        dests.append(jnp.where(keep, topi[k] * CAP + pos, E * CAP))
        wvs.append(jnp.where(keep, wk[:, k:k + 1], 0.0))
    dest_ref[...] = jnp.concatenate(dests, axis=1)
    wv_ref[...] = jnp.concatenate(wvs, axis=1)


def _dispatch_body(T, D, NSLOT, TK,
                   xn_hbm, dest_hbm, wv_hbm, buf_hbm, scale_hbm,
                   dest_ch, wv_ch, scale_v, idx_t, dflat_t, rows_v,
                   sem0, sem1, wsem):
    c = lax.axis_index("c")
    s_id = lax.axis_index("s")
    wid = c * NS + s_id
    n_init = NSLOT // LANES
    n_ch = TK // 2048

    CT = 16
    nck = (T // (NC * NS)) // CT
    tbase = wid * (T // (NC * NS))
    sems = (sem0, sem1)

    def _lstart(j, b):
        pltpu.async_copy(
            xn_hbm.at[pl.ds(tbase + j * CT, CT)], rows_v.at[b], sems[b]
        )

    def _lwait(j, b):
        pltpu.make_async_copy(
            xn_hbm.at[pl.ds(tbase + j * CT, CT)], rows_v.at[b], sems[b]
        ).wait()

    _lstart(0, 0)

    @pl.when((s_id == 0) & (c == 0))
    def _build():
        z_f = jnp.zeros((LANES,), jnp.float32)

        @pl.loop(0, n_init)
        def _init(i):
            scale_v[pl.ds(i * LANES, LANES)] = z_f

        @pl.loop(0, n_ch)
        def _chunk(cb):
            pltpu.sync_copy(dest_hbm.at[pl.ds(cb * 2048, 2048)], dest_ch)
            pltpu.sync_copy(wv_hbm.at[pl.ds(cb * 2048, 2048)], wv_ch)

            @pl.loop(0, 2048 // LANES)
            def _scat(i):
                d16 = dest_ch[pl.ds(i * LANES, LANES)]
                w16 = wv_ch[pl.ds(i * LANES, LANES)]
                plsc.store_scatter(scale_v, [d16], w16)

        pltpu.sync_copy(scale_v, scale_hbm)

    pltpu.sync_copy(dest_hbm.at[pl.ds(tbase * K, (T // (NC * NS)) * K)], dflat_t)
    i16 = lax.iota(jnp.int32, LANES)
    for k in range(K):
        for ch in range(nck):
            g16 = plsc.load_gather(dflat_t, [(ch * CT + i16) * K + k])
            idx_t[k, ch] = g16

    def _scatter(j, b):
        _lwait(j, b)
        for k in range(K):
            pltpu.async_copy(rows_v.at[b], buf_hbm.at[idx_t.at[k, j]], wsem)
        for k in range(K):
            pltpu.make_async_copy(rows_v.at[b], buf_hbm.at[idx_t.at[k, j]], wsem).wait()

    @pl.loop(0, nck - 2, step=2)
    def _disp(j):
        for b in range(2):
            jj = j + b
            _lstart(jj + 1, 1 - b)
            _scatter(jj, b)

    _lstart(nck - 1, 1)
    _scatter(nck - 2, 0)
    _scatter(nck - 1, 1)


def _mlp_body(CAP, D, buf_ref, wg_ref, wu_ref, wd_ref, sc_ref, y_ref):
    w = buf_ref[...].reshape(CAP, D // 2)
    lo = lax.bitcast_convert_type(lax.shift_left(w, 16), jnp.float32)
    hi = lax.bitcast_convert_type(w & jnp.int32(-65536), jnp.float32)
    xg = jnp.concatenate([lo, hi], axis=1)
    g = jnp.dot(xg, wg_ref[0], preferred_element_type=jnp.float32)
    u = jnp.dot(xg, wu_ref[0], preferred_element_type=jnp.float32)
    a = g * jax.nn.sigmoid(g) * u
    y = jnp.dot(a, wd_ref[0], preferred_element_type=jnp.float32)
    scl = sc_ref[...]
    y = jnp.where(scl > 0.0, y * scl, 0.0)
    lo = _round_bf16_bits(y[:, :D // 2])
    hi = _round_bf16_bits(y[:, D // 2:])
    y_ref[...] = lo | lax.shift_left(hi, 16)


def _combine_body(T, D, y_hbm, dest_hbm, res_hbm, out_hbm,
                  idx_l, rows_v, acc_v, sem0, sem1):
    c = lax.axis_index("c")
    s_id = lax.axis_index("s")
    wid = c * NS + s_id
    tpt = T // (NC * NS)
    tbase = wid * tpt
    pltpu.sync_copy(dest_hbm.at[pl.ds(tbase * K, tpt * K)], idx_l)
    CT = 4
    nch = tpt // CT
    sems = (sem0, sem1)

    def _gstart(j, b):
        pltpu.async_copy(
            y_hbm.at[idx_l.at[pl.ds(j * CT * K, CT * K)]], rows_v.at[b], sems[b]
        )

    def _gwait(j, b):
        pltpu.make_async_copy(
            y_hbm.at[idx_l.at[pl.ds(j * CT * K, CT * K)]], rows_v.at[b], sems[b]
        ).wait()

    def _accum(j, b):
        cb = tbase + j * CT
        pltpu.sync_copy(res_hbm.at[pl.ds(cb, CT)], acc_v)
        _gwait(j, b)
        HD = D // 2

        @pl.loop(0, HD // LANES)
        def _vec(v):
            sl = pl.ds(v * LANES, LANES)
            sh = pl.ds(HD + v * LANES, LANES)
            for t in range(CT):
                a_lo = acc_v[t, sl]
                a_hi = acc_v[t, sh]
                for r in range(K):
                    w = rows_v[b, t * K + r, sl]
                    a_lo = a_lo + plsc.bitcast(lax.shift_left(w, 16), jnp.float32)
                    a_hi = a_hi + plsc.bitcast(w & jnp.int32(-65536), jnp.float32)
                acc_v[t, sl] = a_lo
                acc_v[t, sh] = a_hi

        pltpu.sync_copy(acc_v, out_hbm.at[pl.ds(cb, CT)])

    _gstart(0, 0)

    @pl.loop(0, nch - 2, step=2)
    def _chunk(j):
        for b in range(2):
            jj = j + b
            _gstart(jj + 1, 1 - b)
            _accum(jj, b)

    _gstart(nch - 1, 1)
    _accum(nch - 2, 0)
    _accum(nch - 1, 1)


def kernel(hidden_states, rms_weight, router_w, router_b, w_gate, w_up, w_down):
    b, s, d = hidden_states.shape
    T, D = b * s, d
    E = router_w.shape[1]
    F = w_gate.shape[2]
    CAP = T * K // E
    NSLOT = (E + 1) * CAP
    TK = T * K

    x = hidden_states.reshape(T, D)

    NB = 8
    xn, lg = pl.pallas_call(
        functools.partial(_norm_router_body, D // 2),
        grid=(NB,),
        in_specs=[
            pl.BlockSpec((T // NB, D), lambda i: (i, 0)),
            pl.BlockSpec((1, D), lambda i: (0, 0)),
            pl.BlockSpec((D, E), lambda i: (0, 0)),
            pl.BlockSpec((1, E), lambda i: (0, 0)),
        ],
        out_specs=[
            pl.BlockSpec((T // NB, D // 2), lambda i: (i, 0)),
            pl.BlockSpec((T // NB, E), lambda i: (i, 0)),
        ],
        out_shape=[
            jax.ShapeDtypeStruct((T, D // 2), jnp.int32),
            jax.ShapeDtypeStruct((T, E), jnp.float32),
        ],
    )(x, rms_weight.reshape(1, D), router_w, router_b.reshape(1, E))
    xn3 = xn.reshape(T, 8, 128)

    dest, wv = pl.pallas_call(
        functools.partial(_route_body, T, E, CAP),
        out_shape=[
            jax.ShapeDtypeStruct((T, K), jnp.int32),
            jax.ShapeDtypeStruct((T, K), jnp.float32),
        ],
    )(lg)

    dest_flat = dest.reshape(TK)
    wv_flat = wv.reshape(TK)

    mesh = plsc.VectorSubcoreMesh(core_axis_name="c", subcore_axis_name="s")
    buf, scale = pl.kernel(
        functools.partial(_dispatch_body, T, D, NSLOT, TK),
        out_type=[
            jax.ShapeDtypeStruct((NSLOT, 8, 128), jnp.int32),
            jax.ShapeDtypeStruct((NSLOT,), jnp.float32),
        ],
        mesh=mesh,
        scratch_types=[
            pltpu.VMEM((2048,), jnp.int32),
            pltpu.VMEM((2048,), jnp.float32),
            pltpu.VMEM((NSLOT,), jnp.float32),
            pltpu.VMEM((K, 4, 16), jnp.int32),
            pltpu.VMEM((512,), jnp.int32),
            pltpu.VMEM((2, 16, 8, 128), jnp.int32),
            pltpu.SemaphoreType.DMA,
            pltpu.SemaphoreType.DMA,
            pltpu.SemaphoreType.DMA,
        ],
        compiler_params=pltpu.CompilerParams(needs_layout_passes=False),
    )(xn3, dest_flat, wv_flat)

    nblk = E + 1
    y = pl.pallas_call(
        functools.partial(_mlp_body, CAP, D),
        grid=(nblk,),
        in_specs=[
            pl.BlockSpec((CAP, 8, 128), lambda e: (e, 0, 0)),
            pl.BlockSpec((1, D, F), lambda e: (jnp.minimum(e, E - 1), 0, 0)),
            pl.BlockSpec((1, D, F), lambda e: (jnp.minimum(e, E - 1), 0, 0)),
            pl.BlockSpec((1, F, D), lambda e: (jnp.minimum(e, E - 1), 0, 0)),
            pl.BlockSpec((CAP, 1), lambda e: (e, 0)),
        ],
        out_specs=pl.BlockSpec((CAP, D // 2), lambda e: (e, 0)),
        out_shape=jax.ShapeDtypeStruct((NSLOT, D // 2), jnp.int32),
    )(buf, w_gate, w_up, w_down, scale.reshape(NSLOT, 1))

    out = pl.kernel(
        functools.partial(_combine_body, T, D),
        out_type=jax.ShapeDtypeStruct((T, D), jnp.float32),
        mesh=mesh,
        scratch_types=[
            pltpu.VMEM((T // (NC * NS) * K,), jnp.int32),
            pltpu.VMEM((2, 4 * K, D // 2), jnp.int32),
            pltpu.VMEM((4, D), jnp.float32),
            pltpu.SemaphoreType.DMA,
            pltpu.SemaphoreType.DMA,
        ],
        compiler_params=pltpu.CompilerParams(needs_layout_passes=False),
    )(y, dest_flat, x)

    return out.reshape(b, s, d)


# speedup vs baseline: 1.0749x; 1.0487x over previous
import functools

import jax
import jax.numpy as jnp
from jax import lax
from jax.experimental import pallas as pl
from jax.experimental.pallas import tpu as pltpu
from jax.experimental.pallas import tpu_sc as plsc

EPS = 1e-5
K = 8
NC = 2
NS = 16
LANES = 16


def _round_bf16_bits(v):
    bits = lax.bitcast_convert_type(v, jnp.int32)
    rnd = bits + jnp.int32(0x7FFF) + (lax.shift_right_logical(bits, 16) & 1)
    return lax.shift_right_logical(rnd, 16)


def _norm_router_body(HD, x_ref, rw_ref, w_ref, b_ref, xn_ref, lg_ref):
    x = x_ref[...]
    var = jnp.mean(x * x, axis=1, keepdims=True)
    xn = x * lax.rsqrt(var + EPS) * rw_ref[...]
    lo = _round_bf16_bits(xn[:, :HD])
    hi = _round_bf16_bits(xn[:, HD:])
    xn_ref[...] = lo | lax.shift_left(hi, 16)
    lg_ref[...] = (
        jnp.dot(xn, w_ref[...], preferred_element_type=jnp.float32) + b_ref[...]
    )


def _route_body(T, E, CAP, lg_ref, dest_ref, wv_ref):
    scores = jax.nn.sigmoid(lg_ref[...])
    iota_e = lax.broadcasted_iota(jnp.int32, (T, E), 1)
    cur = scores
    topv, topi = [], []
    for _ in range(K):
        m = jnp.max(cur, axis=1, keepdims=True)
        ism = cur == m
        idx = jnp.min(jnp.where(ism, iota_e, E), axis=1, keepdims=True)
        topv.append(m)
        topi.append(idx)
        cur = jnp.where(iota_e == idx, -1.0, cur)
    tv = jnp.concatenate(topv, axis=1)
    wsum = jnp.sum(tv, axis=1, keepdims=True)
    wk = tv / wsum
    cnt = jnp.zeros((T, E), jnp.float32)
    for k in range(K):
        cnt = cnt + (iota_e == topi[k]).astype(jnp.float32)
    CH = 256
    r_i = lax.broadcasted_iota(jnp.int32, (CH, CH), 0)
    c_i = lax.broadcasted_iota(jnp.int32, (CH, CH), 1)
    tril = (c_i < r_i).astype(jnp.float32)
    parts = []
    carry = jnp.zeros((1, E), jnp.float32)
    for c in range(T // CH):
        blk = cnt[c * CH:(c + 1) * CH, :]
        parts.append(carry + jnp.dot(tril, blk, preferred_element_type=jnp.float32))
        carry = carry + jnp.sum(blk, axis=0, keepdims=True)
    prefix = jnp.concatenate(parts, axis=0)
    dests, wvs = [], []
    for k in range(K):
        sel = (iota_e == topi[k]).astype(jnp.float32)
        pos = jnp.sum(prefix * sel, axis=1, keepdims=True).astype(jnp.int32)
        keep = pos < CAP
        dests.append(jnp.where(keep, topi[k] * CAP + pos, E * CAP))
        wvs.append(jnp.where(keep, wk[:, k:k + 1], 0.0))
    dest_ref[...] = jnp.concatenate(dests, axis=1)
    wv_ref[...] = jnp.concatenate(wvs, axis=1)


def _dispatch_body(T, D, NSLOT, TK,
                   xn_hbm, dest_hbm, wv_hbm, buf_hbm, scale_hbm,
                   dest_ch, wv_ch, scale_v, idx_t, dflat_t, rows_v,
                   sem0, sem1, wsem):
    c = lax.axis_index("c")
    s_id = lax.axis_index("s")
    wid = c * NS + s_id
    n_init = NSLOT // LANES
    n_ch = TK // 2048

    CT = 16
    nck = (T // (NC * NS)) // CT
    tbase = wid * (T // (NC * NS))
    sems = (sem0, sem1)

    def _lstart(j, b):
        pltpu.async_copy(
            xn_hbm.at[pl.ds(tbase + j * CT, CT)], rows_v.at[b], sems[b]
        )

    def _lwait(j, b):
        pltpu.make_async_copy(
            xn_hbm.at[pl.ds(tbase + j * CT, CT)], rows_v.at[b], sems[b]
        ).wait()

    _lstart(0, 0)

    @pl.when((s_id == 0) & (c == 0))
    def _build():
        z_f = jnp.zeros((LANES,), jnp.float32)

        @pl.loop(0, n_init)
        def _init(i):
            scale_v[pl.ds(i * LANES, LANES)] = z_f

        @pl.loop(0, n_ch)
        def _chunk(cb):
            pltpu.sync_copy(dest_hbm.at[pl.ds(cb * 2048, 2048)], dest_ch)
            pltpu.sync_copy(wv_hbm.at[pl.ds(cb * 2048, 2048)], wv_ch)

            @pl.loop(0, 2048 // LANES)
            def _scat(i):
                d16 = dest_ch[pl.ds(i * LANES, LANES)]
                w16 = wv_ch[pl.ds(i * LANES, LANES)]
                plsc.store_scatter(scale_v, [d16], w16)

        pltpu.sync_copy(scale_v, scale_hbm)

    pltpu.sync_copy(dest_hbm.at[pl.ds(tbase * K, (T // (NC * NS)) * K)], dflat_t)
    i16 = lax.iota(jnp.int32, LANES)
    for k in range(K):
        for ch in range(nck):
            g16 = plsc.load_gather(dflat_t, [(ch * CT + i16) * K + k])
            idx_t[k, ch] = g16

    def _scatter(j, b):
        _lwait(j, b)
        for k in range(K):
            pltpu.async_copy(rows_v.at[b], buf_hbm.at[idx_t.at[k, j]], wsem)
        for k in range(K):
            pltpu.make_async_copy(rows_v.at[b], buf_hbm.at[idx_t.at[k, j]], wsem).wait()

    @pl.loop(0, nck - 2, step=2)
    def _disp(j):
        for b in range(2):
            jj = j + b
            _lstart(jj + 1, 1 - b)
            _scatter(jj, b)

    _lstart(nck - 1, 1)
    _scatter(nck - 2, 0)
    _scatter(nck - 1, 1)


def _mlp_body(CAP, D, buf_ref, wg_ref, wu_ref, wd_ref, sc_ref, y_ref):
    w = buf_ref[...].reshape(CAP, D // 2)
    lo = lax.bitcast_convert_type(lax.shift_left(w, 16), jnp.float32)
    hi = lax.bitcast_convert_type(w & jnp.int32(-65536), jnp.float32)
    xg = jnp.concatenate([lo, hi], axis=1)
    g = jnp.dot(xg, wg_ref[0], preferred_element_type=jnp.float32)
    u = jnp.dot(xg, wu_ref[0], preferred_element_type=jnp.float32)
    a = g * jax.nn.sigmoid(g) * u
    y = jnp.dot(a, wd_ref[0], preferred_element_type=jnp.float32)
    scl = sc_ref[...]
    y = jnp.where(scl > 0.0, y * scl, 0.0)
    lo = _round_bf16_bits(y[:, :D // 2])
    hi = _round_bf16_bits(y[:, D // 2:])
    y_ref[...] = lo | lax.shift_left(hi, 16)


def _combine_body(T, D, y_hbm, dest_hbm, res_hbm, out_hbm,
                  idx_l, rows_v, acc_v, sem0, sem1, rs0, rs1):
    c = lax.axis_index("c")
    s_id = lax.axis_index("s")
    wid = c * NS + s_id
    tpt = T // (NC * NS)
    tbase = wid * tpt
    pltpu.sync_copy(dest_hbm.at[pl.ds(tbase * K, tpt * K)], idx_l)
    CT = 4
    nch = tpt // CT
    sems = (sem0, sem1)

    def _gstart(j, b):
        pltpu.async_copy(
            y_hbm.at[idx_l.at[pl.ds(j * CT * K, CT * K)]], rows_v.at[b], sems[b]
        )

    def _gwait(j, b):
        pltpu.make_async_copy(
            y_hbm.at[idx_l.at[pl.ds(j * CT * K, CT * K)]], rows_v.at[b], sems[b]
        ).wait()

    rsems = (rs0, rs1)

    def _rstart(j, b):
        pltpu.async_copy(
            res_hbm.at[pl.ds(tbase + j * CT, CT)], acc_v.at[b], rsems[b]
        )

    def _rwait(j, b):
        pltpu.make_async_copy(
            res_hbm.at[pl.ds(tbase + j * CT, CT)], acc_v.at[b], rsems[b]
        ).wait()

    def _accum(j, b):
        cb = tbase + j * CT
        _rwait(j, b)
        _gwait(j, b)
        HD = D // 2

        @pl.loop(0, HD // LANES)
        def _vec(v):
            sl = pl.ds(v * LANES, LANES)
            sh = pl.ds(HD + v * LANES, LANES)
            for t in range(CT):
                a_lo = acc_v[b, t, sl]
                a_hi = acc_v[b, t, sh]
                for r in range(K):
                    w = rows_v[b, t * K + r, sl]
                    a_lo = a_lo + plsc.bitcast(lax.shift_left(w, 16), jnp.float32)
                    a_hi = a_hi + plsc.bitcast(w & jnp.int32(-65536), jnp.float32)
                acc_v[b, t, sl] = a_lo
                acc_v[b, t, sh] = a_hi

        pltpu.sync_copy(acc_v.at[b], out_hbm.at[pl.ds(cb, CT)])

    _gstart(0, 0)
    _rstart(0, 0)

    @pl.loop(0, nch - 2, step=2)
    def _chunk(j):
        for b in range(2):
            jj = j + b
            _gstart(jj + 1, 1 - b)
            _rstart(jj + 1, 1 - b)
            _accum(jj, b)

    _gstart(nch - 1, 1)
    _rstart(nch - 1, 1)
    _accum(nch - 2, 0)
    _accum(nch - 1, 1)


def kernel(hidden_states, rms_weight, router_w, router_b, w_gate, w_up, w_down):
    b, s, d = hidden_states.shape
    T, D = b * s, d
    E = router_w.shape[1]
    F = w_gate.shape[2]
    CAP = T * K // E
    NSLOT = (E + 1) * CAP
    TK = T * K

    x = hidden_states.reshape(T, D)

    NB = 8
    xn, lg = pl.pallas_call(
        functools.partial(_norm_router_body, D // 2),
        grid=(NB,),
        in_specs=[
            pl.BlockSpec((T // NB, D), lambda i: (i, 0)),
            pl.BlockSpec((1, D), lambda i: (0, 0)),
            pl.BlockSpec((D, E), lambda i: (0, 0)),
            pl.BlockSpec((1, E), lambda i: (0, 0)),
        ],
        out_specs=[
            pl.BlockSpec((T // NB, D // 2), lambda i: (i, 0)),
            pl.BlockSpec((T // NB, E), lambda i: (i, 0)),
        ],
        out_shape=[
            jax.ShapeDtypeStruct((T, D // 2), jnp.int32),
            jax.ShapeDtypeStruct((T, E), jnp.float32),
        ],
    )(x, rms_weight.reshape(1, D), router_w, router_b.reshape(1, E))
    xn3 = xn.reshape(T, 8, 128)

    dest, wv = pl.pallas_call(
        functools.partial(_route_body, T, E, CAP),
        out_shape=[
            jax.ShapeDtypeStruct((T, K), jnp.int32),
            jax.ShapeDtypeStruct((T, K), jnp.float32),
        ],
    )(lg)

    dest_flat = dest.reshape(TK)
    wv_flat = wv.reshape(TK)

    mesh = plsc.VectorSubcoreMesh(core_axis_name="c", subcore_axis_name="s")
    buf, scale = pl.kernel(
        functools.partial(_dispatch_body, T, D, NSLOT, TK),
        out_type=[
            jax.ShapeDtypeStruct((NSLOT, 8, 128), jnp.int32),
            jax.ShapeDtypeStruct((NSLOT,), jnp.float32),
        ],
        mesh=mesh,
        scratch_types=[
            pltpu.VMEM((2048,), jnp.int32),
            pltpu.VMEM((2048,), jnp.float32),
            pltpu.VMEM((NSLOT,), jnp.float32),
            pltpu.VMEM((K, 4, 16), jnp.int32),
            pltpu.VMEM((512,), jnp.int32),
            pltpu.VMEM((2, 16, 8, 128), jnp.int32),
            pltpu.SemaphoreType.DMA,
            pltpu.SemaphoreType.DMA,
            pltpu.SemaphoreType.DMA,
        ],
        compiler_params=pltpu.CompilerParams(needs_layout_passes=False),
    )(xn3, dest_flat, wv_flat)

    nblk = E + 1
    y = pl.pallas_call(
        functools.partial(_mlp_body, CAP, D),
        grid=(nblk,),
        in_specs=[
            pl.BlockSpec((CAP, 8, 128), lambda e: (e, 0, 0)),
            pl.BlockSpec((1, D, F), lambda e: (jnp.minimum(e, E - 1), 0, 0)),
            pl.BlockSpec((1, D, F), lambda e: (jnp.minimum(e, E - 1), 0, 0)),
            pl.BlockSpec((1, F, D), lambda e: (jnp.minimum(e, E - 1), 0, 0)),
            pl.BlockSpec((CAP, 1), lambda e: (e, 0)),
        ],
        out_specs=pl.BlockSpec((CAP, D // 2), lambda e: (e, 0)),
        out_shape=jax.ShapeDtypeStruct((NSLOT, D // 2), jnp.int32),
    )(buf, w_gate, w_up, w_down, scale.reshape(NSLOT, 1))

    out = pl.kernel(
        functools.partial(_combine_body, T, D),
        out_type=jax.ShapeDtypeStruct((T, D), jnp.float32),
        mesh=mesh,
        scratch_types=[
            pltpu.VMEM((T // (NC * NS) * K,), jnp.int32),
            pltpu.VMEM((2, 4 * K, D // 2), jnp.int32),
            pltpu.VMEM((2, 4, D), jnp.float32),
            pltpu.SemaphoreType.DMA,
            pltpu.SemaphoreType.DMA,
            pltpu.SemaphoreType.DMA,
            pltpu.SemaphoreType.DMA,
        ],
        compiler_params=pltpu.CompilerParams(needs_layout_passes=False),
    )(y, dest_flat, x)

    return out.reshape(b, s, d)
